# Initial kernel scaffold; baseline (speedup 1.0000x reference)
import jax
import jax.numpy as jnp
from jax.experimental import pallas as pl


def kernel(user_emb, item_emb, adj_values, adj_indices):
    raise NotImplementedError("write your pallas kernel here")


# speedup vs baseline: 3.0691x; 3.0691x over previous
import functools

import jax
import jax.numpy as jnp
from jax import lax
from jax.experimental import pallas as pl
from jax.experimental.pallas import tpu as pltpu
from jax.experimental.pallas import tpu_sc as plsc

USER_N = 5000
N = 10000
E = 320000
D = 128
L = 16
NC = 2
NS = 16
NW = NC * NS
C = 128
CHUNKS = 79
EPW = CHUNKS * C
E_PAD = NW * EPW
ROWS_PER_SUB = 624


_GATHER_DN = lax.GatherDimensionNumbers(
    offset_dims=(), collapsed_slice_dims=(0,), start_index_map=(0,))


def _splat(v16, t):
    idx = jnp.full((L, 1), t, jnp.int32)
    return lax.gather(v16, idx, _GATHER_DN, (1,),
                      mode=lax.GatherScatterMode.PROMISE_IN_BOUNDS)


def _sc_layer_body(ego_hbm, col_hbm, row_hbm, val_hbm, out_hbm,
                   cidx, ridx, vals, rows, acc, sem):
    c = lax.axis_index("c")
    s = lax.axis_index("s")

    zero = jnp.zeros((L,), jnp.float32)

    def _zero_rows(i, carry):
        for j in range(D // L):
            rows[i, pl.ds(j * L, L)] = zero
        return carry

    lax.fori_loop(0, C, _zero_rows, 0)
    for b in range(4):
        pltpu.sync_copy(rows.at[pl.ds(0, C)],
                        acc.at[pl.ds(s * ROWS_PER_SUB + b * C, C)])
    pltpu.sync_copy(rows.at[pl.ds(0, ROWS_PER_SUB - 4 * C)],
                    acc.at[pl.ds(s * ROWS_PER_SUB + 4 * C, ROWS_PER_SUB - 4 * C)])

    @pl.when(s == NS - 1)
    def _zero_tail():
        pltpu.sync_copy(rows.at[pl.ds(0, N - NS * ROWS_PER_SUB)],
                        acc.at[pl.ds(NS * ROWS_PER_SUB, N - NS * ROWS_PER_SUB)])

    plsc.subcore_barrier()

    base0 = (s * NC + c) * EPW

    def _chunk(k, carry):
        base = base0 + k * C
        pltpu.sync_copy(col_hbm.at[pl.ds(base, C)], cidx)
        pltpu.sync_copy(row_hbm.at[pl.ds(base, C)], ridx)
        pltpu.sync_copy(val_hbm.at[pl.ds(base, C)], vals)
        pltpu.async_copy(ego_hbm.at[cidx], rows, sem).wait()

        def _scale16(g, inner):
            v16 = vals[pl.ds(g * L, L)]
            for t in range(L):
                sp = _splat(v16, t)
                e = g * L + t
                for j in range(D // L):
                    rows[e, pl.ds(j * L, L)] = rows[e, pl.ds(j * L, L)] * sp
            return inner

        lax.fori_loop(0, C // L, _scale16, 0)
        pltpu.sync_copy(rows, acc.at[ridx], add=True)
        return carry

    lax.fori_loop(0, CHUNKS, _chunk, 0)
    plsc.subcore_barrier()

    pltpu.sync_copy(acc.at[pl.ds(s * ROWS_PER_SUB, ROWS_PER_SUB)],
                    out_hbm.at[pl.ds(c * N + s * ROWS_PER_SUB, ROWS_PER_SUB)])

    @pl.when(s == NS - 1)
    def _write_tail():
        pltpu.sync_copy(acc.at[pl.ds(NS * ROWS_PER_SUB, N - NS * ROWS_PER_SUB)],
                        out_hbm.at[pl.ds(c * N + NS * ROWS_PER_SUB,
                                         N - NS * ROWS_PER_SUB)])


_sc_layer = functools.partial(
    pl.kernel,
    out_type=jax.ShapeDtypeStruct((NC * N, D), jnp.float32),
    mesh=plsc.VectorSubcoreMesh(core_axis_name="c", subcore_axis_name="s"),
    scratch_types=[
        pltpu.VMEM((C,), jnp.int32),
        pltpu.VMEM((C,), jnp.int32),
        pltpu.VMEM((C,), jnp.float32),
        pltpu.VMEM((C, D), jnp.float32),
        pltpu.VMEM_SHARED((N, D), jnp.float32),
        pltpu.SemaphoreType.DMA,
    ],
)(_sc_layer_body)


def _add_halves_body(a_ref, b_ref, o_ref):
    o_ref[...] = a_ref[...] + b_ref[...]


def _add_halves(p):
    return pl.pallas_call(
        _add_halves_body,
        out_shape=jax.ShapeDtypeStruct((N, D), jnp.float32),
        grid=(10,),
        in_specs=[pl.BlockSpec((N // 10, D), lambda i: (i, 0)),
                  pl.BlockSpec((N // 10, D), lambda i: (i + 10, 0))],
        out_specs=pl.BlockSpec((N // 10, D), lambda i: (i, 0)),
    )(p, p)


def _mean_body(p3a_ref, p3b_ref, e1_ref, e2_ref, o_ref):
    o_ref[...] = (p3a_ref[...] + p3b_ref[...] + e1_ref[...] + e2_ref[...]) * (
        jnp.float32(1.0 / 3.0))


def _user_mean(p3, e1, e2):
    blk = USER_N // 5
    return pl.pallas_call(
        _mean_body,
        out_shape=jax.ShapeDtypeStruct((USER_N, D), jnp.float32),
        grid=(5,),
        in_specs=[
            pl.BlockSpec((blk, D), lambda i: (i, 0)),
            pl.BlockSpec((blk, D), lambda i: (i + N // blk, 0)),
            pl.BlockSpec((blk, D), lambda i: (i, 0)),
            pl.BlockSpec((blk, D), lambda i: (i, 0)),
        ],
        out_specs=pl.BlockSpec((blk, D), lambda i: (i, 0)),
    )(p3, p3, e1, e2)


def kernel(user_emb, item_emb, adj_values, adj_indices):
    ego0 = jnp.concatenate([user_emb, item_emb], axis=0)
    row = adj_indices[0].astype(jnp.int32)
    col = adj_indices[1].astype(jnp.int32)
    pad = E_PAD - E
    zpad_i = jnp.zeros((pad,), jnp.int32)
    colp = jnp.concatenate([col, zpad_i])
    rowp = jnp.concatenate([row, zpad_i])
    valp = jnp.concatenate([adj_values, jnp.zeros((pad,), jnp.float32)])

    p1 = _sc_layer(ego0, colp, rowp, valp)
    e1 = _add_halves(p1)
    p2 = _sc_layer(e1, colp, rowp, valp)
    e2 = _add_halves(p2)
    p3 = _sc_layer(e2, colp, rowp, valp)
    user_out = _user_mean(p3, e1, e2)
    return (user_out, item_emb)


# speedup vs baseline: 5.7264x; 1.8659x over previous
import functools

import jax
import jax.numpy as jnp
from jax import lax
from jax.experimental import pallas as pl
from jax.experimental.pallas import tpu as pltpu
from jax.experimental.pallas import tpu_sc as plsc

USER_N = 5000
N = 10000
E = 320000
D = 128
L = 16
NC = 2
NS = 16
NW = NC * NS
C = 112
CHUNKS = 90
EPW = CHUNKS * C
E_PAD = NW * EPW
ROWS_PER_SUB = 624


_GATHER_DN = lax.GatherDimensionNumbers(
    offset_dims=(), collapsed_slice_dims=(0,), start_index_map=(0,))


def _splat(v16, t):
    idx = jnp.full((L, 1), t, jnp.int32)
    return lax.gather(v16, idx, _GATHER_DN, (1,),
                      mode=lax.GatherScatterMode.PROMISE_IN_BOUNDS)


def _sc_layer_body(ego_hbm, col_hbm, row_hbm, val_hbm, out_hbm,
                   cidx_all, vals_all, ridx_a, ridx_b, rows_a, rows_b, acc,
                   gsem_a, gsem_b, rsem_a, rsem_b):
    c = lax.axis_index("c")
    s = lax.axis_index("s")
    w = s * NC + c

    pltpu.sync_copy(col_hbm.at[pl.ds(w * EPW, EPW)], cidx_all)
    pltpu.sync_copy(val_hbm.at[pl.ds(w * EPW, EPW)], vals_all)

    zero = jnp.zeros((L,), jnp.float32)

    def _zero_rows(i, carry):
        for j in range(D // L):
            rows_a[i, pl.ds(j * L, L)] = zero
        return carry

    lax.fori_loop(0, C, _zero_rows, 0)
    for b in range(5):
        pltpu.sync_copy(rows_a.at[pl.ds(0, C)],
                        acc.at[pl.ds(s * ROWS_PER_SUB + b * C, C)])
    pltpu.sync_copy(rows_a.at[pl.ds(0, ROWS_PER_SUB - 5 * C)],
                    acc.at[pl.ds(s * ROWS_PER_SUB + 5 * C, ROWS_PER_SUB - 5 * C)])

    @pl.when(s == NS - 1)
    def _zero_tail():
        pltpu.sync_copy(rows_a.at[pl.ds(0, N - NS * ROWS_PER_SUB)],
                        acc.at[pl.ds(NS * ROWS_PER_SUB, N - NS * ROWS_PER_SUB)])

    plsc.subcore_barrier()

    base0 = w * EPW
    pltpu.async_copy(row_hbm.at[pl.ds(base0, C)], ridx_a, rsem_a)
    pltpu.async_copy(ego_hbm.at[cidx_all.at[pl.ds(0, C)]], rows_a, gsem_a)

    def _process(k, rows_x, ridx_x, gsem_x, rsem_x,
                 rows_y, ridx_y, gsem_y, rsem_y):
        @pl.when(k + 1 < CHUNKS)
        def _prefetch():
            pltpu.async_copy(row_hbm.at[pl.ds(base0 + (k + 1) * C, C)],
                             ridx_y, rsem_y)
            pltpu.async_copy(ego_hbm.at[cidx_all.at[pl.ds((k + 1) * C, C)]],
                             rows_y, gsem_y)

        pltpu.make_async_copy(ego_hbm.at[cidx_all.at[pl.ds(k * C, C)]],
                              rows_x, gsem_x).wait()

        def _scale16(g, inner):
            v16 = vals_all[pl.ds(k * C + g * L, L)]
            for t in range(L):
                sp = _splat(v16, t)
                e = g * L + t
                for j in range(D // L):
                    rows_x[e, pl.ds(j * L, L)] = rows_x[e, pl.ds(j * L, L)] * sp
            return inner

        lax.fori_loop(0, C // L, _scale16, 0)
        pltpu.make_async_copy(row_hbm.at[pl.ds(base0 + k * C, C)],
                              ridx_x, rsem_x).wait()
        pltpu.sync_copy(rows_x, acc.at[ridx_x], add=True)

    def _chunk(k, carry):
        @pl.when(k % 2 == 0)
        def _even():
            _process(k, rows_a, ridx_a, gsem_a, rsem_a,
                     rows_b, ridx_b, gsem_b, rsem_b)

        @pl.when(k % 2 == 1)
        def _odd():
            _process(k, rows_b, ridx_b, gsem_b, rsem_b,
                     rows_a, ridx_a, gsem_a, rsem_a)

        return carry

    lax.fori_loop(0, CHUNKS, _chunk, 0)
    plsc.subcore_barrier()

    pltpu.sync_copy(acc.at[pl.ds(s * ROWS_PER_SUB, ROWS_PER_SUB)],
                    out_hbm.at[pl.ds(c * N + s * ROWS_PER_SUB, ROWS_PER_SUB)])

    @pl.when(s == NS - 1)
    def _write_tail():
        pltpu.sync_copy(acc.at[pl.ds(NS * ROWS_PER_SUB, N - NS * ROWS_PER_SUB)],
                        out_hbm.at[pl.ds(c * N + NS * ROWS_PER_SUB,
                                         N - NS * ROWS_PER_SUB)])


_sc_layer = functools.partial(
    pl.kernel,
    out_type=jax.ShapeDtypeStruct((NC * N, D), jnp.float32),
    mesh=plsc.VectorSubcoreMesh(core_axis_name="c", subcore_axis_name="s"),
    scratch_types=[
        pltpu.VMEM((EPW,), jnp.int32),
        pltpu.VMEM((EPW,), jnp.float32),
        pltpu.VMEM((C,), jnp.int32),
        pltpu.VMEM((C,), jnp.int32),
        pltpu.VMEM((C, D), jnp.float32),
        pltpu.VMEM((C, D), jnp.float32),
        pltpu.VMEM_SHARED((N, D), jnp.float32),
        pltpu.SemaphoreType.DMA,
        pltpu.SemaphoreType.DMA,
        pltpu.SemaphoreType.DMA,
        pltpu.SemaphoreType.DMA,
    ],
)(_sc_layer_body)


def _add_halves_body(a_ref, b_ref, o_ref):
    o_ref[...] = a_ref[...] + b_ref[...]


def _add_halves(p):
    return pl.pallas_call(
        _add_halves_body,
        out_shape=jax.ShapeDtypeStruct((N, D), jnp.float32),
        grid=(10,),
        in_specs=[pl.BlockSpec((N // 10, D), lambda i: (i, 0)),
                  pl.BlockSpec((N // 10, D), lambda i: (i + 10, 0))],
        out_specs=pl.BlockSpec((N // 10, D), lambda i: (i, 0)),
    )(p, p)


def _mean_body(p3a_ref, p3b_ref, e1_ref, e2_ref, o_ref):
    o_ref[...] = (p3a_ref[...] + p3b_ref[...] + e1_ref[...] + e2_ref[...]) * (
        jnp.float32(1.0 / 3.0))


def _user_mean(p3, e1, e2):
    blk = USER_N // 5
    return pl.pallas_call(
        _mean_body,
        out_shape=jax.ShapeDtypeStruct((USER_N, D), jnp.float32),
        grid=(5,),
        in_specs=[
            pl.BlockSpec((blk, D), lambda i: (i, 0)),
            pl.BlockSpec((blk, D), lambda i: (i + N // blk, 0)),
            pl.BlockSpec((blk, D), lambda i: (i, 0)),
            pl.BlockSpec((blk, D), lambda i: (i, 0)),
        ],
        out_specs=pl.BlockSpec((blk, D), lambda i: (i, 0)),
    )(p3, p3, e1, e2)


def kernel(user_emb, item_emb, adj_values, adj_indices):
    ego0 = jnp.concatenate([user_emb, item_emb], axis=0)
    row = adj_indices[0].astype(jnp.int32)
    col = adj_indices[1].astype(jnp.int32)
    pad = E_PAD - E
    zpad_i = jnp.zeros((pad,), jnp.int32)
    colp = jnp.concatenate([col, zpad_i])
    rowp = jnp.concatenate([row, zpad_i])
    valp = jnp.concatenate([adj_values, jnp.zeros((pad,), jnp.float32)])

    p1 = _sc_layer(ego0, colp, rowp, valp)
    e1 = _add_halves(p1)
    p2 = _sc_layer(e1, colp, rowp, valp)
    e2 = _add_halves(p2)
    p3 = _sc_layer(e2, colp, rowp, valp)
    user_out = _user_mean(p3, e1, e2)
    return (user_out, item_emb)
